# Initial kernel scaffold; baseline (speedup 1.0000x reference)
import jax
import jax.numpy as jnp
from jax.experimental import pallas as pl


def kernel(x, edge_index, edge_attr, batch, W0, b0, W1, b1, W2, b2, gamma0, beta0, gamma1, beta1, gamma2, beta2, prelu_a):
    raise NotImplementedError("write your pallas kernel here")


# speedup vs baseline: 4.2570x; 4.2570x over previous
import functools

import jax
import jax.numpy as jnp
from jax import lax
from jax.experimental import pallas as pl
from jax.experimental.pallas import tpu as pltpu
from jax.experimental.pallas import tpu_sc as plsc

N = 10000
E = 320000
D = 128
G = 64

NC = 2
NS = 16
EPC = E // NC
EPT = EPC // NS
CHUNK = 80
NCHUNK = EPT // CHUNK
RPT = 624


def _sc_msgpass(lin, src, dst, attr, zeros):
  mesh = plsc.VectorSubcoreMesh(
      core_axis_name="c", subcore_axis_name="s",
      num_cores=NC, num_subcores=NS)

  @functools.partial(
      pl.kernel,
      out_type=jax.ShapeDtypeStruct((NC, N, D), jnp.float32),
      mesh=mesh,
      scratch_types=[
          pltpu.VMEM_SHARED((N, D), jnp.float32),
          pltpu.VMEM((CHUNK,), jnp.int32),
          pltpu.VMEM((CHUNK,), jnp.int32),
          pltpu.VMEM((CHUNK,), jnp.float32),
          pltpu.VMEM((CHUNK, D), jnp.float32),
          pltpu.SemaphoreType.DMA,
      ],
  )
  def k(lin_hbm, src_hbm, dst_hbm, attr_hbm, zeros_hbm, out_hbm,
        acc_sh, src_v, dst_v, attr_v, rows_v, sem):
    c = lax.axis_index("c")
    s = lax.axis_index("s")

    @pl.when(s < NS - 1)
    def _():
      pltpu.sync_copy(zeros_hbm.at[pl.ds(s * RPT, RPT)],
                      acc_sh.at[pl.ds(s * RPT, RPT)])

    @pl.when(s == NS - 1)
    def _():
      pltpu.sync_copy(zeros_hbm.at[pl.ds(RPT * (NS - 1), N - RPT * (NS - 1))],
                      acc_sh.at[pl.ds(RPT * (NS - 1), N - RPT * (NS - 1))])

    plsc.subcore_barrier()

    base = c * EPC + s * EPT

    def body(i, carry):
      off = base + i * CHUNK
      pltpu.sync_copy(src_hbm.at[pl.ds(off, CHUNK)], src_v)
      pltpu.sync_copy(dst_hbm.at[pl.ds(off, CHUNK)], dst_v)
      pltpu.sync_copy(attr_hbm.at[pl.ds(off, CHUNK)], attr_v)
      pltpu.async_copy(lin_hbm.at[src_v], rows_v, sem).wait()
      for g in range(CHUNK // 16):
        av = attr_v[pl.ds(g * 16, 16)]
        for t in range(16):
          kk = g * 16 + t
          ab = av[t]
          for j in range(D // 16):
            rows_v[kk, pl.ds(j * 16, 16)] = rows_v[kk, pl.ds(j * 16, 16)] * ab
      pltpu.sync_copy(rows_v, acc_sh.at[dst_v], add=True)
      return carry

    lax.fori_loop(0, NCHUNK, body, 0)
    plsc.subcore_barrier()

    @pl.when(s < NS - 1)
    def _():
      pltpu.sync_copy(acc_sh.at[pl.ds(s * RPT, RPT)],
                      out_hbm.at[c, pl.ds(s * RPT, RPT)])

    @pl.when(s == NS - 1)
    def _():
      pltpu.sync_copy(acc_sh.at[pl.ds(RPT * (NS - 1), N - RPT * (NS - 1))],
                      out_hbm.at[c, pl.ds(RPT * (NS - 1), N - RPT * (NS - 1))])

  return k(lin, src, dst, attr, zeros)


def _tc_first(x, w, b):
  def body(x_ref, w_ref, b_ref, lin_ref):
    lin_ref[...] = jnp.dot(x_ref[...], w_ref[...],
                           preferred_element_type=jnp.float32) + b_ref[...]
  return pl.pallas_call(
      body,
      out_shape=jax.ShapeDtypeStruct((N, D), jnp.float32),
  )(x, w, b)


def _tc_mid(agg, gamma, beta, a, w, b, batch2d):
  def body(agg_ref, g_ref, be_ref, a_ref, w_ref, b_ref, batch_ref,
           lin_ref, pool_ref):
    sm = agg_ref[0] + agg_ref[1]
    av = a_ref[0, 0]
    p = jnp.where(sm >= 0, sm, av * sm)
    mean = jnp.mean(p, axis=0, keepdims=True)
    d = p - mean
    var = jnp.mean(d * d, axis=0, keepdims=True)
    hh = d * lax.rsqrt(var + 1e-5) * g_ref[...] + be_ref[...]
    lin_ref[...] = jnp.dot(hh, w_ref[...],
                           preferred_element_type=jnp.float32) + b_ref[...]
    oh = (jnp.broadcast_to(batch_ref[...], (G, N))
          == lax.broadcasted_iota(jnp.int32, (G, N), 0)).astype(jnp.float32)
    pool_ref[...] = jnp.dot(oh, hh, preferred_element_type=jnp.float32)

  return pl.pallas_call(
      body,
      out_shape=(
          jax.ShapeDtypeStruct((N, D), jnp.float32),
          jax.ShapeDtypeStruct((G, D), jnp.float32),
      ),
  )(agg, gamma, beta, a, w, b, batch2d)


def _tc_last(agg, gamma, beta, a, batch2d):
  def body(agg_ref, g_ref, be_ref, a_ref, batch_ref, pool_ref):
    sm = agg_ref[0] + agg_ref[1]
    av = a_ref[0, 0]
    p = jnp.where(sm >= 0, sm, av * sm)
    mean = jnp.mean(p, axis=0, keepdims=True)
    d = p - mean
    var = jnp.mean(d * d, axis=0, keepdims=True)
    hh = d * lax.rsqrt(var + 1e-5) * g_ref[...] + be_ref[...]
    oh = (jnp.broadcast_to(batch_ref[...], (G, N))
          == lax.broadcasted_iota(jnp.int32, (G, N), 0)).astype(jnp.float32)
    pool_ref[...] = jnp.dot(oh, hh, preferred_element_type=jnp.float32)

  return pl.pallas_call(
      body,
      out_shape=jax.ShapeDtypeStruct((G, D), jnp.float32),
  )(agg, gamma, beta, a, batch2d)


def kernel(x, edge_index, edge_attr, batch, W0, b0, W1, b1, W2, b2,
           gamma0, beta0, gamma1, beta1, gamma2, beta2, prelu_a):
  src = edge_index[0]
  dst = edge_index[1]
  batch2d = batch.reshape(1, N)
  a2d = prelu_a.reshape(1, 1)
  zeros = jnp.zeros((N, D), jnp.float32)
  bs = [b0.reshape(1, D), b1.reshape(1, D), b2.reshape(1, D)]
  gs = [gamma0.reshape(1, D), gamma1.reshape(1, D), gamma2.reshape(1, D)]
  bes = [beta0.reshape(1, D), beta1.reshape(1, D), beta2.reshape(1, D)]

  lin = _tc_first(x, W0, bs[0])
  agg = _sc_msgpass(lin, src, dst, edge_attr, zeros)
  lin, pool0 = _tc_mid(agg, gs[0], bes[0], a2d, W1, bs[1], batch2d)
  agg = _sc_msgpass(lin, src, dst, edge_attr, zeros)
  lin, pool1 = _tc_mid(agg, gs[1], bes[1], a2d, W2, bs[2], batch2d)
  agg = _sc_msgpass(lin, src, dst, edge_attr, zeros)
  pool2 = _tc_last(agg, gs[2], bes[2], a2d, batch2d)

  global_rep = jnp.concatenate([pool0, pool1, pool2], axis=1)
  return (global_rep, pool2)
